# baseline (speedup 1.0000x reference)
import functools
import math

import jax
import jax.numpy as jnp
from jax import lax
from jax.experimental import pallas as pl
from jax.experimental.pallas import tpu as pltpu
from jax.experimental.pallas import tpu_sc as plsc

VOCAB = 1_000_000
D = 64
B = 16384
SCALE = math.sqrt(D)

NC = 2
NS = 16
NW = NC * NS
G = 16
WIN = 128
CHUNK_W = 512
WPW = 244
LPW = WPW * WIN
N_CHUNK = LPW // CHUNK_W
TAIL_LO = 999_936
NSLOT = 16
NBANK = 8


def _body(idx_hbm, tableT_hbm, out_hbm,
          idx_all, my_pos, buf, tailbuf, stag, ctr, sem_in, sem_out):
    wid = lax.axis_index("s") * NC + lax.axis_index("c")
    is_last = wid == NW - 1
    lane_lo = wid * LPW
    lane_hi = jnp.where(is_last, VOCAB, lane_lo + LPW)
    ctr[0] = 0
    ctr[1] = 0

    pltpu.sync_copy(idx_hbm, idx_all)
    iota = lax.iota(jnp.int32, G)

    def start_chunk(c, slot):
        base = lane_lo + c * CHUNK_W
        for c0 in range(D // 8):
            pltpu.async_copy(
                tableT_hbm.at[pl.ds(8 * c0, 8), pl.ds(base, CHUNK_W)],
                buf.at[slot, pl.ds(8 * c0, 8)],
                sem_in,
            )

    start_chunk(0, 0)
    start_chunk(1, 1)

    def sel(g4, cur):
        for k in range(4):
            g = g4 * 4 + k
            v = idx_all[pl.ds(g * G, G)]
            m = (v >= lane_lo) & (v < lane_hi)
            cnt = plsc.all_reduce_population_count(m)[0]

            @pl.when(cnt > 0)
            def _(m=m, g=g, cur=cur):
                plsc.store_compressed(
                    my_pos.at[pl.ds(cur, G)], iota + g * G, mask=m
                )

            cur = cur + cnt
        return cur

    nmine = lax.fori_loop(0, B // G // 4, sel, 0)
    ngrp = (nmine + G - 1) // G

    def do_token(gather_fn, l, pos):
        t = ctr[0]
        slot = jnp.bitwise_and(t, NSLOT - 1)
        bank = jnp.bitwise_and(lax.shift_right_logical(t, 4), NBANK - 1)

        @pl.when((slot == 0) & (t >= NBANK * NSLOT))
        def _():
            pltpu.make_async_copy(
                stag.at[0], out_hbm.at[pl.ds(0, NSLOT)], sem_out
            ).wait()
            ctr[1] = ctr[1] + 1

        lsplat = jnp.full((G,), l, jnp.int32)
        for g3 in range(D // G):
            vals = gather_fn(iota + g3 * G, lsplat)
            stag[bank, slot, pl.ds(g3 * G, G)] = vals * SCALE
        pltpu.async_copy(stag.at[bank, slot], out_hbm.at[pos], sem_out)
        ctr[0] = t + 1

    def scan_window(gather_fn, c_lo, width):
        def grp(g2, carry):
            p = my_pos[pl.ds(g2 * G, G)]
            p = jnp.bitwise_and(p, B - 1)
            v = plsc.load_gather(idx_all, [p])
            valid = iota < (nmine - g2 * G)
            m = valid & (v >= c_lo) & (v < c_lo + width)

            mi = m.astype(jnp.int32)

            @pl.when(plsc.all_reduce_population_count(m)[0] > 0)
            def _():
                for j in range(G):
                    mj = mi[j]
                    vj = v[j]
                    pj = p[j]

                    @pl.when(mj > 0)
                    def _(vj=vj, pj=pj):
                        do_token(gather_fn, vj - c_lo, pj)

            return carry

        lax.fori_loop(0, ngrp, grp, 0)

    trip = jnp.where(is_last, N_CHUNK + 1, N_CHUNK)

    def chunk_loop(c, carry):
        pltpu.make_async_copy(
            tableT_hbm.at[:, pl.ds(0, CHUNK_W)], buf.at[0], sem_in
        ).wait()
        cbsplat = jnp.full((G,), lax.rem(c, 2), jnp.int32)

        def gather_buf(rows, lanes):
            return plsc.load_gather(buf, [cbsplat, rows, lanes])

        scan_window(gather_buf, lane_lo + c * CHUNK_W, CHUNK_W)

        @pl.when(c + 2 < trip)
        def _():
            start_chunk(c + 2, lax.rem(c, 2))

        return carry

    lax.fori_loop(0, trip, chunk_loop, 0)

    @pl.when(is_last)
    def _():
        pltpu.sync_copy(
            tableT_hbm.at[:, pl.ds(TAIL_LO, VOCAB - TAIL_LO)], tailbuf
        )

        def gather_tail(rows, lanes):
            return plsc.load_gather(tailbuf, [rows, lanes])

        scan_window(gather_tail, TAIL_LO, VOCAB - TAIL_LO)

    t = ctr[0]
    d = ctr[1]

    def drain_bank(i, carry):
        pltpu.make_async_copy(
            stag.at[0], out_hbm.at[pl.ds(0, NSLOT)], sem_out
        ).wait()
        return carry

    lax.fori_loop(0, t // NSLOT - d, drain_bank, 0)

    def drain_one(i, carry):
        pltpu.make_async_copy(
            stag.at[0, 0], out_hbm.at[0], sem_out
        ).wait()
        return carry

    lax.fori_loop(0, lax.rem(t, NSLOT), drain_one, 0)


def kernel(token_ids, embedding_table):
    idx = token_ids.astype(jnp.int32)
    table_t = embedding_table.T
    run = functools.partial(
        pl.kernel,
        out_type=jax.ShapeDtypeStruct((B, D), jnp.float32),
        mesh=plsc.VectorSubcoreMesh(core_axis_name="c", subcore_axis_name="s"),
        compiler_params=pltpu.CompilerParams(needs_layout_passes=False),
        scratch_types=[
            pltpu.VMEM((B,), jnp.int32),
            pltpu.VMEM((B + G,), jnp.int32),
            pltpu.VMEM((2, D, CHUNK_W), jnp.float32),
            pltpu.VMEM((D, VOCAB - TAIL_LO), jnp.float32),
            pltpu.VMEM((NBANK, NSLOT, D), jnp.float32),
            pltpu.SMEM((2,), jnp.int32),
            pltpu.SemaphoreType.DMA,
            pltpu.SemaphoreType.DMA,
        ],
    )(_body)
    return run(idx, table_t)


# speedup vs baseline: 1.0020x; 1.0020x over previous
import functools
import math

import jax
import jax.numpy as jnp
from jax import lax
from jax.experimental import pallas as pl
from jax.experimental.pallas import tpu as pltpu
from jax.experimental.pallas import tpu_sc as plsc

VOCAB = 1_000_000
D = 64
B = 16384
SCALE = math.sqrt(D)

NC = 2
NS = 16
NW = NC * NS
G = 16
WIN = 128
CHUNK_W = 512
WPW = 244
LPW = WPW * WIN
N_CHUNK = LPW // CHUNK_W
TAIL_LO = 999_936
NSLOT = 16
NBANK = 2


def _body(idx_hbm, tableT_hbm, out_hbm,
          idx_all, my_pos, buf, tailbuf, stag, ctr, sem_in, sem_out):
    wid = lax.axis_index("s") * NC + lax.axis_index("c")
    is_last = wid == NW - 1
    lane_lo = wid * LPW
    lane_hi = jnp.where(is_last, VOCAB, lane_lo + LPW)
    ctr[0] = 0
    ctr[1] = 0

    pltpu.sync_copy(idx_hbm, idx_all)
    iota = lax.iota(jnp.int32, G)

    def start_chunk(c, slot):
        base = lane_lo + c * CHUNK_W
        for c0 in range(D // 8):
            pltpu.async_copy(
                tableT_hbm.at[pl.ds(8 * c0, 8), pl.ds(base, CHUNK_W)],
                buf.at[slot, pl.ds(8 * c0, 8)],
                sem_in,
            )

    start_chunk(0, 0)
    start_chunk(1, 1)

    def sel(g4, cur):
        for k in range(4):
            g = g4 * 4 + k
            v = idx_all[pl.ds(g * G, G)]
            m = (v >= lane_lo) & (v < lane_hi)
            cnt = plsc.all_reduce_population_count(m)[0]

            @pl.when(cnt > 0)
            def _(m=m, g=g, cur=cur):
                plsc.store_compressed(
                    my_pos.at[pl.ds(cur, G)], iota + g * G, mask=m
                )

            cur = cur + cnt
        return cur

    nmine = lax.fori_loop(0, B // G // 4, sel, 0)
    ngrp = (nmine + G - 1) // G

    def do_token(gather_fn, l, pos):
        t = ctr[0]
        slot = jnp.bitwise_and(t, NSLOT - 1)
        bank = jnp.bitwise_and(lax.shift_right_logical(t, 4), NBANK - 1)

        @pl.when((slot == 0) & (t >= NBANK * NSLOT))
        def _():
            pltpu.make_async_copy(
                stag.at[0], out_hbm.at[pl.ds(0, NSLOT)], sem_out
            ).wait()
            ctr[1] = ctr[1] + 1

        lsplat = jnp.full((G,), l, jnp.int32)
        for g3 in range(D // G):
            vals = gather_fn(iota + g3 * G, lsplat)
            stag[bank, slot, pl.ds(g3 * G, G)] = vals * SCALE
        pltpu.async_copy(stag.at[bank, slot], out_hbm.at[pos], sem_out)
        ctr[0] = t + 1

    def scan_window(gather_fn, c_lo, width):
        def grp(g2, carry):
            p = my_pos[pl.ds(g2 * G, G)]
            p = jnp.bitwise_and(p, B - 1)
            v = plsc.load_gather(idx_all, [p])
            valid = iota < (nmine - g2 * G)
            m = valid & (v >= c_lo) & (v < c_lo + width)

            mi = m.astype(jnp.int32)

            @pl.when(plsc.all_reduce_population_count(m)[0] > 0)
            def _():
                for j in range(G):
                    mj = mi[j]
                    vj = v[j]
                    pj = p[j]

                    @pl.when(mj > 0)
                    def _(vj=vj, pj=pj):
                        do_token(gather_fn, vj - c_lo, pj)

            return carry

        lax.fori_loop(0, ngrp, grp, 0)

    trip = jnp.where(is_last, N_CHUNK + 1, N_CHUNK)

    def chunk_loop(c, carry):
        pltpu.make_async_copy(
            tableT_hbm.at[:, pl.ds(0, CHUNK_W)], buf.at[0], sem_in
        ).wait()
        cbsplat = jnp.full((G,), lax.rem(c, 2), jnp.int32)

        def gather_buf(rows, lanes):
            return plsc.load_gather(buf, [cbsplat, rows, lanes])

        scan_window(gather_buf, lane_lo + c * CHUNK_W, CHUNK_W)

        @pl.when(c + 2 < trip)
        def _():
            start_chunk(c + 2, lax.rem(c, 2))

        return carry

    lax.fori_loop(0, trip, chunk_loop, 0)

    @pl.when(is_last)
    def _():
        pltpu.sync_copy(
            tableT_hbm.at[:, pl.ds(TAIL_LO, VOCAB - TAIL_LO)], tailbuf
        )

        def gather_tail(rows, lanes):
            return plsc.load_gather(tailbuf, [rows, lanes])

        scan_window(gather_tail, TAIL_LO, VOCAB - TAIL_LO)

    t = ctr[0]
    d = ctr[1]

    def drain_bank(i, carry):
        pltpu.make_async_copy(
            stag.at[0], out_hbm.at[pl.ds(0, NSLOT)], sem_out
        ).wait()
        return carry

    lax.fori_loop(0, t // NSLOT - d, drain_bank, 0)

    def drain_one(i, carry):
        pltpu.make_async_copy(
            stag.at[0, 0], out_hbm.at[0], sem_out
        ).wait()
        return carry

    lax.fori_loop(0, lax.rem(t, NSLOT), drain_one, 0)


def kernel(token_ids, embedding_table):
    idx = token_ids.astype(jnp.int32)
    table_t = embedding_table.T
    run = functools.partial(
        pl.kernel,
        out_type=jax.ShapeDtypeStruct((B, D), jnp.float32),
        mesh=plsc.VectorSubcoreMesh(core_axis_name="c", subcore_axis_name="s"),
        compiler_params=pltpu.CompilerParams(needs_layout_passes=False),
        scratch_types=[
            pltpu.VMEM((B,), jnp.int32),
            pltpu.VMEM((B + G,), jnp.int32),
            pltpu.VMEM((2, D, CHUNK_W), jnp.float32),
            pltpu.VMEM((D, VOCAB - TAIL_LO), jnp.float32),
            pltpu.VMEM((NBANK, NSLOT, D), jnp.float32),
            pltpu.SMEM((2,), jnp.int32),
            pltpu.SemaphoreType.DMA,
            pltpu.SemaphoreType.DMA,
        ],
    )(_body)
    return run(idx, table_t)


# speedup vs baseline: 1.3910x; 1.3883x over previous
import functools
import math

import jax
import jax.numpy as jnp
from jax import lax
from jax.experimental import pallas as pl
from jax.experimental.pallas import tpu as pltpu
from jax.experimental.pallas import tpu_sc as plsc

VOCAB = 1_000_000
D = 64
B = 16384
SCALE = math.sqrt(D)

NC = 2
NS = 16
NW = NC * NS
G = 16
WIN = 128
CHUNK_W = 512
WPW = 244
LPW = WPW * WIN
N_CHUNK = LPW // CHUNK_W
TAIL_LO = 999_936
NSLOT = 16
NBANK = 2


def _body(idx_hbm, tableT_hbm, out_hbm,
          idx_all, my_ids, my_pos, buf, tailbuf, stag, poslist, ctr,
          sem_in, sem_out):
    wid = lax.axis_index("s") * NC + lax.axis_index("c")
    is_last = wid == NW - 1
    lane_lo = wid * LPW
    lane_hi = jnp.where(is_last, VOCAB, lane_lo + LPW)
    ctr[0] = 0
    ctr[1] = 0

    pltpu.sync_copy(idx_hbm, idx_all)
    iota = lax.iota(jnp.int32, G)

    def start_chunk(c, slot):
        base = lane_lo + c * CHUNK_W
        for c0 in range(D // 8):
            pltpu.async_copy(
                tableT_hbm.at[pl.ds(8 * c0, 8), pl.ds(base, CHUNK_W)],
                buf.at[slot, pl.ds(8 * c0, 8)],
                sem_in,
            )

    start_chunk(0, 0)
    start_chunk(1, 1)

    def sel(g4, cur):
        for k in range(4):
            g = g4 * 4 + k
            v = idx_all[pl.ds(g * G, G)]
            m = (v >= lane_lo) & (v < lane_hi)
            cnt = plsc.all_reduce_population_count(m)[0]

            @pl.when(cnt > 0)
            def _(v=v, m=m, g=g, cur=cur):
                plsc.store_compressed(my_ids.at[pl.ds(cur, G)], v, mask=m)
                plsc.store_compressed(
                    my_pos.at[pl.ds(cur, G)], iota + g * G, mask=m
                )

            cur = cur + cnt
        return cur

    nmine = lax.fori_loop(0, B // G // 4, sel, 0)
    ngrp = (nmine + G - 1) // G

    lane0 = iota == 0

    def do_token(gather_fn, l, pos):
        t = ctr[0]
        slot = jnp.bitwise_and(t, NSLOT - 1)
        bank = jnp.bitwise_and(lax.shift_right_logical(t, 4), NBANK - 1)

        @pl.when((slot == 0) & (t >= NBANK * NSLOT))
        def _():
            pltpu.make_async_copy(
                stag.at[0], out_hbm.at[poslist.at[0]], sem_out
            ).wait()
            ctr[1] = ctr[1] + 1

        lsplat = jnp.full((G,), l, jnp.int32)
        for g3 in range(D // G):
            vals = gather_fn(iota + g3 * G, lsplat)
            stag[bank, slot, pl.ds(g3 * G, G)] = vals * SCALE
        plsc.store_scatter(
            poslist,
            [jnp.full((G,), bank, jnp.int32), jnp.full((G,), slot, jnp.int32)],
            jnp.full((G,), pos, jnp.int32),
            mask=lane0,
        )
        ctr[2] = pos

        @pl.when(slot == NSLOT - 1)
        def _():
            pltpu.async_copy(
                stag.at[bank], out_hbm.at[poslist.at[bank]], sem_out
            )

        ctr[0] = t + 1

    def scan_window(gather_fn, c_lo, width):
        def grp(g2, carry):
            v = my_ids[pl.ds(g2 * G, G)]
            p = my_pos[pl.ds(g2 * G, G)]
            valid = iota < (nmine - g2 * G)
            m = valid & (v >= c_lo) & (v < c_lo + width)

            mi = m.astype(jnp.int32)

            @pl.when(plsc.all_reduce_population_count(m)[0] > 0)
            def _():
                for j in range(G):
                    mj = mi[j]
                    vj = v[j]
                    pj = p[j]

                    @pl.when(mj > 0)
                    def _(vj=vj, pj=pj):
                        do_token(gather_fn, vj - c_lo, pj)

            return carry

        lax.fori_loop(0, ngrp, grp, 0)

    trip = jnp.where(is_last, N_CHUNK + 1, N_CHUNK)

    def chunk_loop(c, carry):
        pltpu.make_async_copy(
            tableT_hbm.at[:, pl.ds(0, CHUNK_W)], buf.at[0], sem_in
        ).wait()
        cbsplat = jnp.full((G,), lax.rem(c, 2), jnp.int32)

        def gather_buf(rows, lanes):
            return plsc.load_gather(buf, [cbsplat, rows, lanes])

        scan_window(gather_buf, lane_lo + c * CHUNK_W, CHUNK_W)

        @pl.when(c + 2 < trip)
        def _():
            start_chunk(c + 2, lax.rem(c, 2))

        return carry

    lax.fori_loop(0, trip, chunk_loop, 0)

    @pl.when(is_last)
    def _():
        pltpu.sync_copy(
            tableT_hbm.at[:, pl.ds(TAIL_LO, VOCAB - TAIL_LO)], tailbuf
        )

        def gather_tail(rows, lanes):
            return plsc.load_gather(tailbuf, [rows, lanes])

        scan_window(gather_tail, TAIL_LO, VOCAB - TAIL_LO)

    t = ctr[0]
    r = jnp.bitwise_and(t, NSLOT - 1)
    fbank = jnp.bitwise_and(lax.shift_right_logical(t, 4), NBANK - 1)

    @pl.when(r > 0)
    def _():
        lastpos = ctr[2]
        plsc.store_scatter(
            poslist,
            [jnp.full((G,), fbank, jnp.int32), iota],
            jnp.full((G,), lastpos, jnp.int32),
            mask=iota >= r,
        )
        for j in range(NSLOT):
            @pl.when(j >= r)
            def _(j=j):
                for g3 in range(D // G):
                    sl = pl.ds(g3 * G, G)
                    stag[fbank, j, sl] = stag[fbank, r - 1, sl]

        pltpu.async_copy(
            stag.at[fbank], out_hbm.at[poslist.at[fbank]], sem_out
        )

    fired = t // NSLOT + jnp.where(r > 0, 1, 0)
    d = ctr[1]

    def drain_bank(i, carry):
        pltpu.make_async_copy(
            stag.at[0], out_hbm.at[poslist.at[0]], sem_out
        ).wait()
        return carry

    lax.fori_loop(0, fired - d, drain_bank, 0)


def kernel(token_ids, embedding_table):
    idx = token_ids.astype(jnp.int32)
    table_t = embedding_table.T
    run = functools.partial(
        pl.kernel,
        out_type=jax.ShapeDtypeStruct((B, 2 * D), jnp.float32),
        mesh=plsc.VectorSubcoreMesh(core_axis_name="c", subcore_axis_name="s"),
        compiler_params=pltpu.CompilerParams(needs_layout_passes=False),
        scratch_types=[
            pltpu.VMEM((B,), jnp.int32),
            pltpu.VMEM((B + G,), jnp.int32),
            pltpu.VMEM((B + G,), jnp.int32),
            pltpu.VMEM((2, D, CHUNK_W), jnp.float32),
            pltpu.VMEM((D, VOCAB - TAIL_LO), jnp.float32),
            pltpu.VMEM((NBANK, NSLOT, 2 * D), jnp.float32),
            pltpu.VMEM((NBANK, NSLOT), jnp.int32),
            pltpu.SMEM((4,), jnp.int32),
            pltpu.SemaphoreType.DMA,
            pltpu.SemaphoreType.DMA,
        ],
    )(_body)
    return run(idx, table_t)[:, :D]


# speedup vs baseline: 2.2355x; 1.6071x over previous
import functools
import math

import jax
import jax.numpy as jnp
from jax import lax
from jax.experimental import pallas as pl
from jax.experimental.pallas import tpu as pltpu
from jax.experimental.pallas import tpu_sc as plsc

VOCAB = 1_000_000
D = 64
B = 16384
SCALE = math.sqrt(D)

NC = 2
NS = 16
NW = NC * NS
G = 16
WIN = 128
CHUNK_W = 512
WPW = 244
LPW = WPW * WIN
N_CHUNK = LPW // CHUNK_W
TAIL_LO = 999_936
NSLOT = 16
NBANK = 2


def _body(idx_hbm, tableT_hbm, out_hbm,
          idx_all, my_ids, my_pos, buf, tailbuf, stag, poslist,
          tmp_ids, tmp_pos, ctr, sem_in, sem_out):
    wid = lax.axis_index("s") * NC + lax.axis_index("c")
    is_last = wid == NW - 1
    lane_lo = wid * LPW
    lane_hi = jnp.where(is_last, VOCAB, lane_lo + LPW)
    ctr[0] = 0
    ctr[1] = 0

    pltpu.sync_copy(idx_hbm, idx_all)
    iota = lax.iota(jnp.int32, G)

    def start_chunk(c, slot):
        base = lane_lo + c * CHUNK_W
        for c0 in range(D // 8):
            pltpu.async_copy(
                tableT_hbm.at[pl.ds(8 * c0, 8), pl.ds(base, CHUNK_W)],
                buf.at[slot, pl.ds(8 * c0, 8)],
                sem_in,
            )

    start_chunk(0, 0)
    start_chunk(1, 1)

    def sel(g4, cur):
        for k in range(4):
            g = g4 * 4 + k
            v = idx_all[pl.ds(g * G, G)]
            m = (v >= lane_lo) & (v < lane_hi)
            cnt = plsc.all_reduce_population_count(m)[0]

            @pl.when(cnt > 0)
            def _(v=v, m=m, g=g, cur=cur):
                plsc.store_compressed(my_ids.at[pl.ds(cur, G)], v, mask=m)
                plsc.store_compressed(
                    my_pos.at[pl.ds(cur, G)], iota + g * G, mask=m
                )

            cur = cur + cnt
        return cur

    nmine = lax.fori_loop(0, B // G // 4, sel, 0)
    ngrp = (nmine + G - 1) // G

    lane0 = iota == 0

    def do_token(gather_fn, l, pos):
        t = ctr[0]
        slot = jnp.bitwise_and(t, NSLOT - 1)
        bank = jnp.bitwise_and(lax.shift_right_logical(t, 4), NBANK - 1)

        @pl.when((slot == 0) & (t >= NBANK * NSLOT))
        def _():
            pltpu.make_async_copy(
                stag.at[0], out_hbm.at[poslist.at[0]], sem_out
            ).wait()
            ctr[1] = ctr[1] + 1

        lsplat = jnp.full((G,), l, jnp.int32)
        for g3 in range(D // G):
            vals = gather_fn(iota + g3 * G, lsplat)
            stag[bank, slot, pl.ds(g3 * G, G)] = vals * SCALE
        plsc.store_scatter(
            poslist,
            [jnp.full((G,), bank, jnp.int32), jnp.full((G,), slot, jnp.int32)],
            jnp.full((G,), pos, jnp.int32),
            mask=lane0,
        )
        ctr[2] = pos

        @pl.when(slot == NSLOT - 1)
        def _():
            pltpu.async_copy(
                stag.at[bank], out_hbm.at[poslist.at[bank]], sem_out
            )

        ctr[0] = t + 1

    def scan_window(gather_fn, c_lo, width):
        def grp(g2, carry):
            v = my_ids[pl.ds(g2 * G, G)]
            p = my_pos[pl.ds(g2 * G, G)]
            valid = iota < (nmine - g2 * G)
            m = valid & (v >= c_lo) & (v < c_lo + width)
            cnt = plsc.all_reduce_population_count(m)[0]

            @pl.when(cnt > 0)
            def _():
                plsc.store_compressed(tmp_ids.at[pl.ds(0, G)], v, mask=m)
                plsc.store_compressed(tmp_pos.at[pl.ds(0, G)], p, mask=m)

                def each(j, c2):
                    vj = tmp_ids[pl.ds(j, G)][0]
                    pj = tmp_pos[pl.ds(j, G)][0]
                    do_token(gather_fn, vj - c_lo, pj)
                    return c2

                lax.fori_loop(0, cnt, each, 0)

            return carry

        lax.fori_loop(0, ngrp, grp, 0)

    trip = jnp.where(is_last, N_CHUNK + 1, N_CHUNK)

    def chunk_loop(c, carry):
        pltpu.make_async_copy(
            tableT_hbm.at[:, pl.ds(0, CHUNK_W)], buf.at[0], sem_in
        ).wait()
        cbsplat = jnp.full((G,), lax.rem(c, 2), jnp.int32)

        def gather_buf(rows, lanes):
            return plsc.load_gather(buf, [cbsplat, rows, lanes])

        scan_window(gather_buf, lane_lo + c * CHUNK_W, CHUNK_W)

        @pl.when(c + 2 < trip)
        def _():
            start_chunk(c + 2, lax.rem(c, 2))

        return carry

    lax.fori_loop(0, trip, chunk_loop, 0)

    @pl.when(is_last)
    def _():
        pltpu.sync_copy(
            tableT_hbm.at[:, pl.ds(TAIL_LO, VOCAB - TAIL_LO)], tailbuf
        )

        def gather_tail(rows, lanes):
            return plsc.load_gather(tailbuf, [rows, lanes])

        scan_window(gather_tail, TAIL_LO, VOCAB - TAIL_LO)

    t = ctr[0]
    r = jnp.bitwise_and(t, NSLOT - 1)
    fbank = jnp.bitwise_and(lax.shift_right_logical(t, 4), NBANK - 1)

    @pl.when(r > 0)
    def _():
        lastpos = ctr[2]
        plsc.store_scatter(
            poslist,
            [jnp.full((G,), fbank, jnp.int32), iota],
            jnp.full((G,), lastpos, jnp.int32),
            mask=iota >= r,
        )
        for j in range(NSLOT):
            @pl.when(j >= r)
            def _(j=j):
                for g3 in range(D // G):
                    sl = pl.ds(g3 * G, G)
                    stag[fbank, j, sl] = stag[fbank, r - 1, sl]

        pltpu.async_copy(
            stag.at[fbank], out_hbm.at[poslist.at[fbank]], sem_out
        )

    fired = t // NSLOT + jnp.where(r > 0, 1, 0)
    d = ctr[1]

    def drain_bank(i, carry):
        pltpu.make_async_copy(
            stag.at[0], out_hbm.at[poslist.at[0]], sem_out
        ).wait()
        return carry

    lax.fori_loop(0, fired - d, drain_bank, 0)


def kernel(token_ids, embedding_table):
    idx = token_ids.astype(jnp.int32)
    table_t = embedding_table.T
    run = functools.partial(
        pl.kernel,
        out_type=jax.ShapeDtypeStruct((B, 2 * D), jnp.float32),
        mesh=plsc.VectorSubcoreMesh(core_axis_name="c", subcore_axis_name="s"),
        compiler_params=pltpu.CompilerParams(needs_layout_passes=False),
        scratch_types=[
            pltpu.VMEM((B,), jnp.int32),
            pltpu.VMEM((B + G,), jnp.int32),
            pltpu.VMEM((B + G,), jnp.int32),
            pltpu.VMEM((2, D, CHUNK_W), jnp.float32),
            pltpu.VMEM((D, VOCAB - TAIL_LO), jnp.float32),
            pltpu.VMEM((NBANK, NSLOT, 2 * D), jnp.float32),
            pltpu.VMEM((NBANK, NSLOT), jnp.int32),
            pltpu.VMEM((2 * G,), jnp.int32),
            pltpu.VMEM((2 * G,), jnp.int32),
            pltpu.SMEM((4,), jnp.int32),
            pltpu.SemaphoreType.DMA,
            pltpu.SemaphoreType.DMA,
        ],
    )(_body)
    return run(idx, table_t)[:, :D]
